# baseline (speedup 1.0000x reference)
import functools

import jax
import jax.numpy as jnp
from jax import lax
from jax.experimental import pallas as pl
from jax.experimental.pallas import tpu as pltpu
from jax.experimental.pallas import tpu_sc as plsc

BATCH = 16384
EMB_K = 64
_INFO = plsc.get_sparse_core_info()
NC, NS, L = _INFO.num_cores, _INFO.num_subcores, _INFO.num_lanes
NW = NC * NS
B_PER_W = BATCH // NW
IDX_CHUNK = 128
N_CHUNKS = B_PER_W // IDX_CHUNK
N_GROUPS = B_PER_W // L
K_CH = EMB_K // L


def _mf_dot(u_idx, v_idx, W, H):
    mesh = plsc.VectorSubcoreMesh(core_axis_name="c", subcore_axis_name="s")

    @functools.partial(
        pl.kernel,
        mesh=mesh,
        out_type=jax.ShapeDtypeStruct((NW, B_PER_W), jnp.float32),
        compiler_params=pltpu.CompilerParams(use_tc_tiling_on_sc=False),
        scratch_types=[
            pltpu.VMEM((N_CHUNKS, IDX_CHUNK), jnp.int32),
            pltpu.VMEM((N_CHUNKS, IDX_CHUNK), jnp.int32),
            pltpu.VMEM((B_PER_W, EMB_K), jnp.float32),
            pltpu.VMEM((B_PER_W, EMB_K), jnp.float32),
            pltpu.VMEM((B_PER_W,), jnp.float32),
            pltpu.VMEM((2 * L,), jnp.float32),
            pltpu.SemaphoreType.DMA,
            pltpu.SemaphoreType.DMA,
        ],
    )
    def k(u_idx_hbm, v_idx_hbm, w_hbm, h_hbm, out_hbm,
          idx_u, idx_v, u_rows, v_rows, out_v, tree_buf, sem_u, sem_v):
        c = lax.axis_index("c")
        s = lax.axis_index("s")
        wid = s * NC + c

        pltpu.sync_copy(u_idx_hbm.at[wid], idx_u)
        pltpu.sync_copy(v_idx_hbm.at[wid], idx_v)

        copies = []
        for j in range(N_CHUNKS):
            dst = pl.ds(j * IDX_CHUNK, IDX_CHUNK)
            copies.append(pltpu.async_copy(
                w_hbm.at[idx_u.at[j]], u_rows.at[dst], sem_u))
            copies.append(pltpu.async_copy(
                h_hbm.at[idx_v.at[j]], v_rows.at[dst], sem_v))
        for cp in copies:
            cp.wait()

        iota16 = lax.iota(jnp.int32, L)
        tree_buf[pl.ds(L, L)] = jnp.zeros((L,), jnp.float32)

        def group(g, carry):
            acc = jnp.zeros((L,), jnp.float32)
            for r in range(L):
                row = g * L + r
                p = u_rows[row, pl.ds(0, L)] * v_rows[row, pl.ds(0, L)]
                for cch in range(1, K_CH):
                    sl = pl.ds(cch * L, L)
                    p = p + u_rows[row, sl] * v_rows[row, sl]
                tree_buf[pl.ds(0, L)] = p
                t = p
                for sh in (8, 4, 2, 1):
                    t = t + tree_buf[pl.ds(sh, L)]
                    tree_buf[pl.ds(0, L)] = t
                s = t[0]
                acc = jnp.where(iota16 == r, lax.broadcast(s, (L,)), acc)
            out_v[pl.ds(g * L, L)] = acc
            return carry

        lax.fori_loop(0, N_GROUPS, group, 0)
        pltpu.sync_copy(out_v, out_hbm.at[wid])

    return k(u_idx, v_idx, W, H)


def kernel(x, W, H, W_pre, H_pre, W_eps, H_eps):
    xi = x.astype(jnp.int32)
    u_idx = xi[:, 0].reshape(NW, N_CHUNKS, IDX_CHUNK)
    v_idx = xi[:, 1].reshape(NW, N_CHUNKS, IDX_CHUNK)
    out = _mf_dot(u_idx, v_idx, W, H)
    return out.reshape(BATCH)


# speedup vs baseline: 1.3203x; 1.3203x over previous
import functools

import jax
import jax.numpy as jnp
from jax import lax
from jax.experimental import pallas as pl
from jax.experimental.pallas import tpu as pltpu
from jax.experimental.pallas import tpu_sc as plsc

BATCH = 16384
EMB_K = 64
_INFO = plsc.get_sparse_core_info()
NC, NS, L = _INFO.num_cores, _INFO.num_subcores, _INFO.num_lanes
NW = NC * NS
B_PER_W = BATCH // NW
CHUNK = 128
N_CHUNKS = B_PER_W // CHUNK
G_PER_CHUNK = CHUNK // L
K_CH = EMB_K // L


def _mf_dot(u_idx, v_idx, W, H):
    mesh = plsc.VectorSubcoreMesh(core_axis_name="c", subcore_axis_name="s")

    @functools.partial(
        pl.kernel,
        mesh=mesh,
        out_type=jax.ShapeDtypeStruct((NW, B_PER_W), jnp.float32),
        scratch_types=[
            pltpu.VMEM((B_PER_W,), jnp.int32),
            pltpu.VMEM((B_PER_W,), jnp.int32),
            pltpu.VMEM((CHUNK, EMB_K), jnp.float32),
            pltpu.VMEM((CHUNK, EMB_K), jnp.float32),
            pltpu.VMEM((CHUNK, EMB_K), jnp.float32),
            pltpu.VMEM((CHUNK, EMB_K), jnp.float32),
            pltpu.VMEM((B_PER_W,), jnp.float32),
            pltpu.VMEM((2 * L,), jnp.float32),
            pltpu.SemaphoreType.DMA,
            pltpu.SemaphoreType.DMA,
        ],
    )
    def k(u_idx_hbm, v_idx_hbm, w_hbm, h_hbm, out_hbm,
          idx_u_v, idx_v_v,
          u_buf0, u_buf1, v_buf0, v_buf1,
          out_v, tree_buf, sem_u, sem_v):
        c = lax.axis_index("c")
        s = lax.axis_index("s")
        wid = s * NC + c

        pltpu.sync_copy(u_idx_hbm.at[wid], idx_u_v)
        pltpu.sync_copy(v_idx_hbm.at[wid], idx_v_v)

        iota16 = lax.iota(jnp.int32, L)
        tree_buf[pl.ds(L, L)] = jnp.zeros((L,), jnp.float32)

        u_bufs = (u_buf0, u_buf1)
        v_bufs = (v_buf0, v_buf1)

        def fire(chunk):
            ub = u_bufs[chunk % 2]
            vb = v_bufs[chunk % 2]
            base = chunk * CHUNK

            def body(g16, carry):
                off = g16 * L
                uiv = idx_u_v[pl.ds(base + off, L)]
                viv = idx_v_v[pl.ds(base + off, L)]
                for lane in range(L):
                    dst = pl.ds(off + lane, 1)
                    pltpu.async_copy(
                        w_hbm.at[pl.ds(uiv[lane], 1)], ub.at[dst], sem_u)
                    pltpu.async_copy(
                        h_hbm.at[pl.ds(viv[lane], 1)], vb.at[dst], sem_v)
                return carry

            lax.fori_loop(0, CHUNK // L, body, 0)

        def drain(chunk):
            pltpu.make_async_copy(
                w_hbm.at[pl.ds(0, CHUNK)], u_bufs[chunk % 2], sem_u).wait()
            pltpu.make_async_copy(
                h_hbm.at[pl.ds(0, CHUNK)], v_bufs[chunk % 2], sem_v).wait()

        def compute(chunk):
            ub = u_bufs[chunk % 2]
            vb = v_bufs[chunk % 2]
            base = chunk * CHUNK

            def group(g, carry):
                acc = jnp.zeros((L,), jnp.float32)
                for r in range(L):
                    row = g * L + r
                    p = ub[row, pl.ds(0, L)] * vb[row, pl.ds(0, L)]
                    for cch in range(1, K_CH):
                        sl = pl.ds(cch * L, L)
                        p = p + ub[row, sl] * vb[row, sl]
                    tree_buf[pl.ds(0, L)] = p
                    t = p
                    for sh in (8, 4, 2, 1):
                        t = t + tree_buf[pl.ds(sh, L)]
                        tree_buf[pl.ds(0, L)] = t
                    acc = jnp.where(
                        iota16 == r, lax.broadcast(t[0], (L,)), acc)
                out_v[pl.ds(base + g * L, L)] = acc
                return carry

            lax.fori_loop(0, G_PER_CHUNK, group, 0)

        fire(0)
        fire(1)
        for chunk in range(N_CHUNKS):
            drain(chunk)
            compute(chunk)
            if chunk + 2 < N_CHUNKS:
                fire(chunk + 2)

        pltpu.sync_copy(out_v, out_hbm.at[wid])

    return k(u_idx, v_idx, W, H)


def kernel(x, W, H, W_pre, H_pre, W_eps, H_eps):
    xi = x.astype(jnp.int32)
    u_idx = xi[:, 0].reshape(NW, B_PER_W)
    v_idx = xi[:, 1].reshape(NW, B_PER_W)
    out = _mf_dot(u_idx, v_idx, W, H)
    return out.reshape(BATCH)
